# Initial kernel scaffold; baseline (speedup 1.0000x reference)
import jax
import jax.numpy as jnp
from jax.experimental import pallas as pl


def kernel(link_state, states_graph_ids, states_first, states_second, states_num_edges, W_msg, b_msg, gru_kernel, gru_rkernel, gru_bias, W_r1, b_r1, W_r2, b_r2, W_r3, b_r3):
    raise NotImplementedError("write your pallas kernel here")


# speedup vs baseline: 6.8961x; 6.8961x over previous
import functools

import jax
import jax.numpy as jnp
from jax import lax
from jax.experimental import pallas as pl
from jax.experimental.pallas import tpu as pltpu
from jax.experimental.pallas import tpu_sc as plsc

_N = 10000
_E = 640000
_D = 120
_DP = 128
_T = 5
_G = 16

_NC = 2
_NS = 16
_NW = _NC * _NS
_EPW = _E // _NW
_CH = 80
_NCHUNK = _EPW // _CH
_NP = 10240
_RPT = _NP // _NS

_SELU_ALPHA = 1.6732632423543772
_SELU_SCALE = 1.0507009873554805

_BLK = 1000
_NBLK = _N // _BLK


def _selu(x):
    return _SELU_SCALE * jnp.where(
        x > 0, x, _SELU_ALPHA * (jnp.exp(x) - 1.0))


def _edge_body(a_hbm, b_hbm, fst_hbm, sec_hbm, out_hbm,
               acc, fst_v, sec_v, arows, brows, sem):
    c = lax.axis_index("c")
    s = lax.axis_index("s")
    wid = c * _NS + s

    zero16 = jnp.zeros((16,), jnp.float32)

    def _zrow(r, carry):
        for j in range(_DP // 16):
            arows[r, pl.ds(j * 16, 16)] = zero16
        return carry

    lax.fori_loop(0, _CH, _zrow, 0)
    rbase = s * _RPT
    for k in range(_RPT // _CH):
        pltpu.sync_copy(arows, acc.at[pl.ds(rbase + k * _CH, _CH)])
    plsc.subcore_barrier()

    ebase = wid * _EPW

    def _chunk(k, carry):
        off = ebase + k * _CH
        pltpu.sync_copy(fst_hbm.at[pl.ds(off, _CH)], fst_v)
        pltpu.sync_copy(sec_hbm.at[pl.ds(off, _CH)], sec_v)
        cp_a = pltpu.async_copy(a_hbm.at[fst_v], arows, sem)
        cp_b = pltpu.async_copy(b_hbm.at[sec_v], brows, sem)
        cp_a.wait()
        cp_b.wait()

        def _row(r, inner):
            for j in range(_DP // 16):
                sl = pl.ds(j * 16, 16)
                x = arows[r, sl] + brows[r, sl]
                arows[r, sl] = _selu(x)
            return inner

        lax.fori_loop(0, _CH, _row, 0)
        pltpu.sync_copy(arows, acc.at[sec_v], add=True)
        return carry

    lax.fori_loop(0, _NCHUNK, _chunk, 0)
    plsc.subcore_barrier()
    pltpu.sync_copy(acc.at[pl.ds(rbase, _RPT)],
                    out_hbm.at[c, pl.ds(rbase, _RPT)])


@functools.cache
def _edge_pass_kernel():
    return pl.kernel(
        _edge_body,
        out_type=jax.ShapeDtypeStruct((_NC, _NP, _DP), jnp.float32),
        mesh=plsc.VectorSubcoreMesh(core_axis_name="c", subcore_axis_name="s",
                                    num_cores=_NC, num_subcores=_NS),
        scratch_types=[
            pltpu.VMEM_SHARED((_NP, _DP), jnp.float32),
            pltpu.VMEM((_CH,), jnp.int32),
            pltpu.VMEM((_CH,), jnp.int32),
            pltpu.VMEM((_CH, _DP), jnp.float32),
            pltpu.VMEM((_CH, _DP), jnp.float32),
            pltpu.SemaphoreType.DMA,
        ],
    )


def _msg0_body(h_ref, w1_ref, w2_ref, bm_ref, a_ref, b_ref):
    h = h_ref[...]
    a_ref[...] = jnp.dot(h, w1_ref[...], preferred_element_type=jnp.float32)
    b_ref[...] = (jnp.dot(h, w2_ref[...], preferred_element_type=jnp.float32)
                  + bm_ref[...])


def _gru_body(p_ref, h_ref, kc_ref, rkc_ref, bx_ref, br_ref,
              w1_ref, w2_ref, bm_ref, hn_ref, a_ref, b_ref):
    x = p_ref[0] + p_ref[1]
    h = h_ref[...]
    mx = jnp.dot(x, kc_ref[...], preferred_element_type=jnp.float32) + bx_ref[...]
    mh = jnp.dot(h, rkc_ref[...], preferred_element_type=jnp.float32) + br_ref[...]
    z = jax.nn.sigmoid(mx[:, :_DP] + mh[:, :_DP])
    r = jax.nn.sigmoid(mx[:, _DP:2 * _DP] + mh[:, _DP:2 * _DP])
    hh = jnp.tanh(mx[:, 2 * _DP:] + r * mh[:, 2 * _DP:])
    hn = z * h + (1.0 - z) * hh
    hn_ref[...] = hn
    a_ref[...] = jnp.dot(hn, w1_ref[...], preferred_element_type=jnp.float32)
    b_ref[...] = (jnp.dot(hn, w2_ref[...], preferred_element_type=jnp.float32)
                  + bm_ref[...])


def _readout_body(h_ref, gid_ref, wr1_ref, wr2_ref, wr3_ref,
                  br1_ref, br2_ref, br3_ref, out_ref):
    h = h_ref[...]
    ids = gid_ref[...]
    rows = []
    for g in range(_G):
        m = (ids == g).astype(jnp.float32)
        rows.append(jnp.sum(h * m, axis=0, keepdims=True))
    ec = jnp.concatenate(rows, axis=0)
    r1 = _selu(jnp.dot(ec, wr1_ref[...], preferred_element_type=jnp.float32)
               + br1_ref[...])
    r2 = _selu(jnp.dot(r1, wr2_ref[...], preferred_element_type=jnp.float32)
               + br2_ref[...])
    out_ref[...] = (jnp.dot(r2, wr3_ref[...], preferred_element_type=jnp.float32)
                    + br3_ref[...])


def _row_spec(shape):
    return pl.BlockSpec(shape, lambda i: (i,) + (0,) * (len(shape) - 1))


def _full_spec(shape):
    return pl.BlockSpec(shape, lambda i: (0,) * len(shape))


_msg0 = pl.pallas_call(
    _msg0_body,
    grid=(_NBLK,),
    in_specs=[_row_spec((_BLK, _DP)), _full_spec((_DP, _DP)),
              _full_spec((_DP, _DP)), _full_spec((1, _DP))],
    out_specs=[_row_spec((_BLK, _DP)), _row_spec((_BLK, _DP))],
    out_shape=[jax.ShapeDtypeStruct((_N, _DP), jnp.float32)] * 2,
)

_gru = pl.pallas_call(
    _gru_body,
    grid=(_NBLK,),
    in_specs=[
        pl.BlockSpec((_NC, _BLK, _DP), lambda i: (0, i, 0)),
        _row_spec((_BLK, _DP)),
        _full_spec((_DP, 3 * _DP)), _full_spec((_DP, 3 * _DP)),
        _full_spec((1, 3 * _DP)), _full_spec((1, 3 * _DP)),
        _full_spec((_DP, _DP)), _full_spec((_DP, _DP)), _full_spec((1, _DP)),
    ],
    out_specs=[_row_spec((_BLK, _DP))] * 3,
    out_shape=[jax.ShapeDtypeStruct((_N, _DP), jnp.float32)] * 3,
)

_readout = pl.pallas_call(
    _readout_body,
    out_shape=jax.ShapeDtypeStruct((_G, _DP), jnp.float32),
)


def _pad_mat(w):
    return jnp.pad(w, ((0, _DP - w.shape[0]), (0, _DP - w.shape[1])))


def _pad_vec(v):
    return jnp.pad(v, (0, _DP - v.shape[0])).reshape(1, _DP)


def kernel(link_state, states_graph_ids, states_first, states_second,
           states_num_edges, W_msg, b_msg, gru_kernel, gru_rkernel, gru_bias,
           W_r1, b_r1, W_r2, b_r2, W_r3, b_r3):
    f32 = jnp.float32
    h = jnp.pad(link_state.astype(f32), ((0, 0), (0, _DP - _D)))
    w1 = _pad_mat(W_msg[:_D])
    w2 = _pad_mat(W_msg[_D:])
    bm = _pad_vec(b_msg)
    kc = jnp.concatenate(
        [_pad_mat(gru_kernel[:, i * _D:(i + 1) * _D]) for i in range(3)], axis=1)
    rkc = jnp.concatenate(
        [_pad_mat(gru_rkernel[:, i * _D:(i + 1) * _D]) for i in range(3)], axis=1)
    bx = jnp.concatenate(
        [_pad_vec(gru_bias[0, i * _D:(i + 1) * _D]) for i in range(3)], axis=1)
    br = jnp.concatenate(
        [_pad_vec(gru_bias[1, i * _D:(i + 1) * _D]) for i in range(3)], axis=1)
    wr1 = _pad_mat(W_r1)
    wr2 = _pad_mat(W_r2)
    wr3 = _pad_mat(W_r3)
    br1 = _pad_vec(b_r1)
    br2 = _pad_vec(b_r2)
    br3 = _pad_vec(b_r3)
    fst = states_first.astype(jnp.int32)
    sec = states_second.astype(jnp.int32)
    gid = states_graph_ids.astype(jnp.int32).reshape(_N, 1)

    a, b = _msg0(h, w1, w2, bm)
    edge_pass = _edge_pass_kernel()
    for _ in range(_T):
        parts = edge_pass(a, b, fst, sec)
        h, a, b = _gru(parts, h, kc, rkc, bx, br, w1, w2, bm)
    out = _readout(h, gid, wr1, wr2, wr3, br1, br2, br3)
    return out[:, 0:1]
